# baseline (speedup 1.0000x reference)
import functools

import jax
import jax.numpy as jnp
from jax import lax
from jax.experimental import pallas as pl
from jax.experimental.pallas import tpu as pltpu
from jax.experimental.pallas import tpu_sc as plsc

_TINY = 1.1754943508222875e-38


def _make_stream_kernel(V, VS, C, K, G):
    rows = 2 * K

    def _stream(ids_smem, unif_smem, t_hbm, d_hbm, d_ref, g_ref, besti_ref,
                bestv_ref, acc_ref, t_vmem, t_sem, tg, dg, g_sem):
        i = pl.program_id(0)
        slot = lax.rem(i, 2)

        def start_copies(step, s):
            pltpu.make_async_copy(
                t_hbm.at[2 * step, 0:K, 0:VS], t_vmem.at[s, 0:K, :],
                t_sem.at[s, 0]).start()
            pltpu.make_async_copy(
                t_hbm.at[2 * step + 1, 0:K, 0:VS],
                t_vmem.at[s, K : 2 * K, :], t_sem.at[s, 1]).start()

        @pl.when(i == 0)
        def _prologue():
            start_copies(0, 0)

        @pl.when(i + 1 < G)
        def _prefetch():
            start_copies(i + 1, lax.rem(i + 1, 2))

        for r in range(rows):
            b, kk = divmod(r, K)
            tid_s = ids_smem[0, b, kk]
            grp = pl.multiple_of((tid_s // 128) * 128, 128)
            pltpu.make_async_copy(
                t_hbm.at[2 * i + b, kk, pl.ds(grp, 128)], tg.at[r],
                g_sem.at[r, 0]).start()
            pltpu.make_async_copy(
                d_hbm.at[2 * i + b, kk, pl.ds(grp, 128)], dg.at[r],
                g_sem.at[r, 1]).start()

        pltpu.make_async_copy(
            t_hbm.at[2 * i, 0:K, 0:VS], t_vmem.at[slot, 0:K, :],
            t_sem.at[slot, 0]).wait()
        pltpu.make_async_copy(
            t_hbm.at[2 * i + 1, 0:K, 0:VS], t_vmem.at[slot, K : 2 * K, :],
            t_sem.at[slot, 1]).wait()

        best_v = jnp.full((rows, 1), -jnp.inf, jnp.float32)
        best_i = jnp.zeros((rows, 1), jnp.int32)
        for c in range(0, VS, C):
            cc = min(C, VS - c)
            t8 = t_vmem[slot, :, c : c + cc]
            d8 = jnp.concatenate(
                [d_ref[0, :, c : c + cc], d_ref[1, :, c : c + cc]], axis=0)
            g8 = jnp.concatenate(
                [g_ref[0, :, c : c + cc], g_ref[1, :, c : c + cc]], axis=0)
            score = jnp.log(jnp.maximum(t8 - d8, _TINY)) + g8
            m = jnp.max(score, axis=1, keepdims=True)
            lane = jax.lax.broadcasted_iota(jnp.int32, (rows, cc), 1)
            loc = jnp.min(jnp.where(score == m, lane, V), axis=1,
                          keepdims=True)
            upd = m > best_v
            best_v = jnp.where(upd, m, best_v)
            best_i = jnp.where(upd, c + loc, best_i)
        besti_ref[0] = best_i
        bestv_ref[0] = best_v

        lane128 = jax.lax.broadcasted_iota(jnp.int32, (1, 128), 1)
        subl = jax.lax.broadcasted_iota(jnp.int32, (rows, 1), 0)
        acc = jnp.zeros((rows, 1), jnp.int32)
        for r in range(rows):
            b, kk = divmod(r, K)
            tid_s = ids_smem[0, b, kk]
            grp = pl.multiple_of((tid_s // 128) * 128, 128)
            pltpu.make_async_copy(
                t_hbm.at[2 * i + b, kk, pl.ds(grp, 128)], tg.at[r],
                g_sem.at[r, 0]).wait()
            pltpu.make_async_copy(
                d_hbm.at[2 * i + b, kk, pl.ds(grp, 128)], dg.at[r],
                g_sem.at[r, 1]).wait()
            msk = lane128 == (tid_s - grp)
            sel_t = jnp.sum(jnp.where(msk, tg[r : r + 1, :], 0.0), axis=1,
                            keepdims=True)
            sel_d = jnp.sum(jnp.where(msk, dg[r : r + 1, :], 0.0), axis=1,
                            keepdims=True)
            a = jnp.where(
                unif_smem[0, b, kk] < jnp.minimum(sel_t / sel_d, 1.0), 1, 0
            ).astype(jnp.int32)
            acc = jnp.where(subl == r, a, acc)
        acc_ref[0] = acc

    return _stream


def _make_sc_tail_kernel(V, VS, B, K):
    TAIL = V - VS
    NCHUNK = TAIL // 16
    ROWS = B * K
    mesh = plsc.VectorSubcoreMesh(core_axis_name="c", subcore_axis_name="s")
    NC, NS = 2, 16
    NW = NC * NS
    RPW = ROWS // NW

    @functools.partial(
        pl.kernel,
        out_type=[
            jax.ShapeDtypeStruct((ROWS, 16), jnp.float32),
            jax.ShapeDtypeStruct((ROWS, 16), jnp.int32),
        ],
        mesh=mesh,
        scratch_types=[
            pltpu.VMEM((TAIL,), jnp.float32),
            pltpu.VMEM((TAIL,), jnp.float32),
            pltpu.VMEM((TAIL,), jnp.float32),
            pltpu.VMEM((16,), jnp.float32),
            pltpu.VMEM((16,), jnp.int32),
        ],
    )
    def _sc_tail(t_hbm, d_hbm, g_hbm, outv_hbm, outi_hbm, tb, db, gb, vb, ib):
        wid = lax.axis_index("s") * NC + lax.axis_index("c")
        lane = lax.iota(jnp.int32, 16)
        for rr in range(RPW):
            row = wid * RPW + rr
            b = lax.div(row, K)
            kk = lax.rem(row, K)
            pltpu.sync_copy(t_hbm.at[b, kk, pl.ds(VS, TAIL)], tb)
            pltpu.sync_copy(d_hbm.at[b, kk, pl.ds(VS, TAIL)], db)
            pltpu.sync_copy(g_hbm.at[b, kk, pl.ds(VS, TAIL)], gb)

            def body(ci, carry):
                bv, bi = carry
                off = ci * 16
                t16 = tb[pl.ds(off, 16)]
                d16 = db[pl.ds(off, 16)]
                g16 = gb[pl.ds(off, 16)]
                s16 = jnp.maximum(t16 - d16, _TINY) * jnp.exp(g16)
                upd = s16 > bv
                return (jnp.where(upd, s16, bv),
                        jnp.where(upd, VS + off + lane, bi))

            bv, bi = lax.fori_loop(
                0, NCHUNK, body,
                (jnp.full((16,), -1.0, jnp.float32),
                 jnp.zeros((16,), jnp.int32)))
            vb[...] = bv
            ib[...] = bi
            pltpu.sync_copy(vb, outv_hbm.at[row])
            pltpu.sync_copy(ib, outi_hbm.at[row])

    return _sc_tail


def _make_epilogue(K):
    def _epi(ids_ref, bonus_ref, besti_ref, bestv_ref, acc_ref, scv_ref,
             sci_ref, out_ref):
        b = ids_ref.shape[0]
        cols = []
        for kk in range(K):
            scv = scv_ref[:, 16 * kk : 16 * (kk + 1)]
            sci = sci_ref[:, 16 * kk : 16 * (kk + 1)]
            scm = jnp.max(scv, axis=1, keepdims=True)
            sc_idx = jnp.min(jnp.where(scv == scm, sci, jnp.int32(2**30)),
                             axis=1, keepdims=True)
            sc_log = jnp.log(scm)
            tc_v = bestv_ref[:, kk : kk + 1]
            tc_i = besti_ref[:, kk : kk + 1]
            cols.append(jnp.where(tc_v >= sc_log, tc_i, sc_idx))
        best = jnp.concatenate(cols, axis=1)

        kidx = jax.lax.broadcasted_iota(jnp.int32, (b, K), 1)
        limits = jnp.min(jnp.where(acc_ref[...] == 0, kidx, K), axis=1,
                         keepdims=True)
        out_k = jnp.where(kidx < limits, ids_ref[...], -1)
        bonus_col = jnp.where(out_k[:, K - 1 : K] != -1, bonus_ref[...], -1)
        out_k = jnp.where(kidx == limits, best, out_k)
        out_ref[:, :K] = out_k
        out_ref[:, K:] = bonus_col

    return _epi


@jax.jit
def kernel(target_with_bonus_probs, bonus_token_ids, draft_probs,
           draft_token_ids, uniform_rand, gumbel_noise):
    B, K, V = draft_probs.shape
    VS = 87040
    C = 1024
    G = B // 2
    rows = 2 * K
    ids3 = draft_token_ids.reshape(G, 2, K)
    unif3 = uniform_rand.reshape(G, 2, K)

    besti, bestv, acc = pl.pallas_call(
        _make_stream_kernel(V, VS, C, K, G),
        grid=(G,),
        in_specs=[
            pl.BlockSpec((1, 2, K), lambda i: (i, 0, 0),
                         memory_space=pltpu.SMEM),
            pl.BlockSpec((1, 2, K), lambda i: (i, 0, 0),
                         memory_space=pltpu.SMEM),
            pl.BlockSpec(memory_space=pl.ANY),
            pl.BlockSpec(memory_space=pl.ANY),
            pl.BlockSpec((2, K, VS), lambda i: (i, 0, 0)),
            pl.BlockSpec((2, K, VS), lambda i: (i, 0, 0)),
        ],
        out_specs=[
            pl.BlockSpec((1, rows, 1), lambda i: (i, 0, 0)),
            pl.BlockSpec((1, rows, 1), lambda i: (i, 0, 0)),
            pl.BlockSpec((1, rows, 1), lambda i: (i, 0, 0)),
        ],
        out_shape=[
            jax.ShapeDtypeStruct((G, rows, 1), jnp.int32),
            jax.ShapeDtypeStruct((G, rows, 1), jnp.float32),
            jax.ShapeDtypeStruct((G, rows, 1), jnp.int32),
        ],
        scratch_shapes=[
            pltpu.VMEM((2, rows, VS), jnp.float32),
            pltpu.SemaphoreType.DMA((2, 2)),
            pltpu.VMEM((rows, 128), jnp.float32),
            pltpu.VMEM((rows, 128), jnp.float32),
            pltpu.SemaphoreType.DMA((rows, 2)),
        ],
        compiler_params=pltpu.CompilerParams(
            dimension_semantics=("arbitrary",),
        ),
    )(ids3, unif3, target_with_bonus_probs, draft_probs, draft_probs,
      gumbel_noise)

    scv, sci = _make_sc_tail_kernel(V, VS, B, K)(
        target_with_bonus_probs, draft_probs, gumbel_noise)

    out = pl.pallas_call(
        _make_epilogue(K),
        out_shape=jax.ShapeDtypeStruct((B, K + 1), jnp.int32),
    )(draft_token_ids, bonus_token_ids, besti.reshape(B, K),
      bestv.reshape(B, K), acc.reshape(B, K), scv.reshape(B, K * 16),
      sci.reshape(B, K * 16))
    return out


# speedup vs baseline: 1.1861x; 1.1861x over previous
import jax
import jax.numpy as jnp
from jax.experimental import pallas as pl
from jax.experimental.pallas import tpu as pltpu

_TINY = 1.1754943508222875e-38


def _make_stream_kernel(V, C, K, G):
    rows = 2 * K
    NBUF = 3

    def _stream(ids_smem, unif_smem, t_hbm, d_hbm, g_hbm, besti_ref, acc_ref,
                t_vmem, d_vmem, g_vmem, sem):
        i = pl.program_id(0)
        slot = jax.lax.rem(i, NBUF)

        def copies(step, s):
            out = []
            for half in range(2):
                b = 2 * step + half
                rr = half * K
                out.append(pltpu.make_async_copy(
                    t_hbm.at[b, 0:K, :], t_vmem.at[s, rr : rr + K, :],
                    sem.at[s, half]))
                out.append(pltpu.make_async_copy(
                    d_hbm.at[b, :, :], d_vmem.at[s, rr : rr + K, :],
                    sem.at[s, 2 + half]))
                out.append(pltpu.make_async_copy(
                    g_hbm.at[b, :, :], g_vmem.at[s, rr : rr + K, :],
                    sem.at[s, 4 + half]))
            return out

        @pl.when(i == 0)
        def _prologue():
            for cp in copies(0, 0) + copies(1, 1):
                cp.start()

        @pl.when(i + 2 < G)
        def _prefetch():
            for cp in copies(i + 2, jax.lax.rem(i + 2, NBUF)):
                cp.start()

        for cp in copies(i, slot):
            cp.wait()

        best_v = jnp.full((rows, 1), -jnp.inf, jnp.float32)
        best_i = jnp.zeros((rows, 1), jnp.int32)
        for c in range(0, V, C):
            cc = min(C, V - c)
            t8 = t_vmem[slot, :, c : c + cc]
            d8 = d_vmem[slot, :, c : c + cc]
            g8 = g_vmem[slot, :, c : c + cc]
            score = jnp.log(jnp.maximum(t8 - d8, _TINY)) + g8
            m = jnp.max(score, axis=1, keepdims=True)
            lane = jax.lax.broadcasted_iota(jnp.int32, (rows, cc), 1)
            loc = jnp.min(jnp.where(score == m, lane, V), axis=1,
                          keepdims=True)
            upd = m > best_v
            best_v = jnp.where(upd, m, best_v)
            best_i = jnp.where(upd, c + loc, best_i)
        besti_ref[0] = best_i

        lane128 = jax.lax.broadcasted_iota(jnp.int32, (1, 128), 1)
        subl = jax.lax.broadcasted_iota(jnp.int32, (rows, 1), 0)
        acc = jnp.zeros((rows, 1), jnp.int32)
        for r in range(rows):
            b, kk = divmod(r, K)
            tid_s = ids_smem[0, b, kk]
            grp = pl.multiple_of((tid_s // 128) * 128, 128)
            tv = t_vmem[slot, r : r + 1, pl.ds(grp, 128)]
            dv = d_vmem[slot, r : r + 1, pl.ds(grp, 128)]
            msk = lane128 == (tid_s - grp)
            sel_t = jnp.sum(jnp.where(msk, tv, 0.0), axis=1, keepdims=True)
            sel_d = jnp.sum(jnp.where(msk, dv, 0.0), axis=1, keepdims=True)
            a = jnp.where(
                unif_smem[0, b, kk] < jnp.minimum(sel_t / sel_d, 1.0), 1, 0
            ).astype(jnp.int32)
            acc = jnp.where(subl == r, a, acc)
        acc_ref[0] = acc

    return _stream


def _epilogue(ids_ref, bonus_ref, besti_ref, acc_ref, out_ref):
    b, k = ids_ref.shape
    kidx = jax.lax.broadcasted_iota(jnp.int32, (b, k), 1)
    limits = jnp.min(jnp.where(acc_ref[...] == 0, kidx, k), axis=1,
                     keepdims=True)
    out_k = jnp.where(kidx < limits, ids_ref[...], -1)
    bonus_col = jnp.where(out_k[:, k - 1 : k] != -1, bonus_ref[...], -1)
    out_k = jnp.where(kidx == limits, besti_ref[...], out_k)
    out_ref[:, :k] = out_k
    out_ref[:, k:] = bonus_col


@jax.jit
def kernel(target_with_bonus_probs, bonus_token_ids, draft_probs,
           draft_token_ids, uniform_rand, gumbel_noise):
    B, K, V = draft_probs.shape
    C = 1024
    G = B // 2
    rows = 2 * K
    ids3 = draft_token_ids.reshape(G, 2, K)
    unif3 = uniform_rand.reshape(G, 2, K)
    besti, acc = pl.pallas_call(
        _make_stream_kernel(V, C, K, G),
        grid=(G,),
        in_specs=[
            pl.BlockSpec((1, 2, K), lambda i: (i, 0, 0),
                         memory_space=pltpu.SMEM),
            pl.BlockSpec((1, 2, K), lambda i: (i, 0, 0),
                         memory_space=pltpu.SMEM),
            pl.BlockSpec(memory_space=pl.ANY),
            pl.BlockSpec(memory_space=pl.ANY),
            pl.BlockSpec(memory_space=pl.ANY),
        ],
        out_specs=[
            pl.BlockSpec((1, rows, 1), lambda i: (i, 0, 0)),
            pl.BlockSpec((1, rows, 1), lambda i: (i, 0, 0)),
        ],
        out_shape=[
            jax.ShapeDtypeStruct((G, rows, 1), jnp.int32),
            jax.ShapeDtypeStruct((G, rows, 1), jnp.int32),
        ],
        scratch_shapes=[
            pltpu.VMEM((3, rows, V), jnp.float32),
            pltpu.VMEM((3, rows, V), jnp.float32),
            pltpu.VMEM((3, rows, V), jnp.float32),
            pltpu.SemaphoreType.DMA((3, 6)),
        ],
        compiler_params=pltpu.CompilerParams(
            dimension_semantics=("arbitrary",),
        ),
    )(ids3, unif3, target_with_bonus_probs, draft_probs, gumbel_noise)

    out = pl.pallas_call(
        _epilogue,
        out_shape=jax.ShapeDtypeStruct((B, K + 1), jnp.int32),
    )(draft_token_ids, bonus_token_ids, besti.reshape(B, K),
      acc.reshape(B, K))
    return out
